# Initial kernel scaffold; baseline (speedup 1.0000x reference)
import jax
import jax.numpy as jnp
from jax.experimental import pallas as pl


def kernel(pij, i, j, table):
    raise NotImplementedError("write your pallas kernel here")


# speedup vs baseline: 4.8222x; 4.8222x over previous
import functools

import jax
import jax.numpy as jnp
from jax import lax
from jax.experimental import pallas as pl
from jax.experimental.pallas import tpu as pltpu
from jax.experimental.pallas import tpu_sc as plsc

_N_POINTS = 1000000
_N_DIM = 16
_NC = 2
_NS = 16
_NW = _NC * _NS
_ROWS_PER_W = _N_POINTS // _NW
_CHUNK = 3125
_NCHUNK = _ROWS_PER_W // _CHUNK
_NACC = 4


def _sc_body(table_hbm, idx_hbm, partials_hbm, rows_hbm,
             idx_v, rows_v, buf0, buf1, acc_v, sem0, sem1, gsem):
  cid = lax.axis_index("c")
  sid = lax.axis_index("s")
  wid = sid * _NC + cid
  base = wid * (_ROWS_PER_W * _N_DIM)

  pltpu.sync_copy(idx_hbm, idx_v)
  pltpu.async_copy(table_hbm.at[idx_v], rows_v, gsem).wait()
  ti = rows_v[pl.ds(0, _N_DIM)]

  bufs = (buf0, buf1)
  sems = (sem0, sem1)
  cwords = _CHUNK * _N_DIM

  def start(c):
    return pltpu.async_copy(
        table_hbm.at[pl.ds(base + c * cwords, cwords)], bufs[c % 2],
        sems[c % 2])

  inflight = start(0)
  accs = tuple(jnp.zeros((_N_DIM,), jnp.float32) for _ in range(_NACC))
  for c in range(_NCHUNK):
    inflight.wait()
    if c + 1 < _NCHUNK:
      inflight = start(c + 1)
    buf = bufs[c % 2]

    @pl.loop(0, _CHUNK // _NACC, init_carry=accs)
    def accs(r, carry):
      off = pl.multiple_of(r * (_NACC * _N_DIM), 8)
      out = []
      for b in range(_NACC):
        x = buf[pl.ds(off + b * _N_DIM, _N_DIM)]
        d = x - ti
        out.append(carry[b] + jnp.exp(-(d * d)))
      return tuple(out)

  total = accs[0]
  for b in range(1, _NACC):
    total = total + accs[b]
  acc_v[...] = total
  pltpu.sync_copy(acc_v, partials_hbm.at[wid])

  @pl.when(wid == 0)
  def _():
    pltpu.sync_copy(rows_v, rows_hbm)


@functools.partial(
    pl.kernel,
    out_type=(
        jax.ShapeDtypeStruct((_NW, _N_DIM), jnp.float32),
        jax.ShapeDtypeStruct((2 * _N_DIM,), jnp.float32),
    ),
    mesh=plsc.VectorSubcoreMesh(core_axis_name="c", subcore_axis_name="s"),
    scratch_types=(
        pltpu.VMEM((2 * _N_DIM,), jnp.int32),
        pltpu.VMEM((2 * _N_DIM,), jnp.float32),
        pltpu.VMEM((_CHUNK * _N_DIM,), jnp.float32),
        pltpu.VMEM((_CHUNK * _N_DIM,), jnp.float32),
        pltpu.VMEM((_N_DIM,), jnp.float32),
        pltpu.SemaphoreType.DMA,
        pltpu.SemaphoreType.DMA,
        pltpu.SemaphoreType.DMA,
    ),
)
def _sc_reduce(table_hbm, idx_hbm, partials_hbm, rows_hbm, *scratch):
  _sc_body(table_hbm, idx_hbm, partials_hbm, rows_hbm, *scratch)


def _tc_epilogue(pij_ref, rows_ref, partials_ref, out_ref):
  q_sum = jnp.sum(partials_ref[...]) - jnp.float32(_N_DIM)
  d = rows_ref[0:1, :] - rows_ref[1:2, :]
  p = pij_ref[...]
  t = p * (jnp.log(p) + d * d + jnp.log(q_sum))
  out_ref[...] = jnp.sum(t).reshape(1, 1)


def kernel(pij, i, j, table):
  lane = jnp.arange(_N_DIM, dtype=jnp.int32)
  idx = jnp.concatenate([i[0] * _N_DIM + lane, j[0] * _N_DIM + lane])
  partials, rows = _sc_reduce(table.reshape(-1), idx)
  out = pl.pallas_call(
      _tc_epilogue,
      out_shape=jax.ShapeDtypeStruct((1, 1), jnp.float32),
  )(pij.reshape(1, _N_DIM), rows.reshape(2, _N_DIM), partials)
  return out[0, 0]


# speedup vs baseline: 34.4562x; 7.1453x over previous
import functools

import jax
import jax.numpy as jnp
from jax import lax
from jax.experimental import pallas as pl
from jax.experimental.pallas import tpu as pltpu
from jax.experimental.pallas import tpu_sc as plsc

_N_POINTS = 1000000
_N_DIM = 16
_NC = 2
_NS = 16
_NW = _NC * _NS
_LANE = 128
_NT_FULL = _N_POINTS // _LANE
_TAIL = _N_POINTS - _NT_FULL * _LANE
_CHT = 12
_CHW = _CHT * _LANE
_NCH = _NT_FULL // _CHT
_CPW = -(-_NCH // _NW)
_FULL_W = _NCH - _NW * (_CPW - 1)


def _sc_body(tt_hbm, ij_hbm, partials_hbm, rows_hbm,
             idx_v, tile_i, tile_j, rows_v, buf0, buf1, acc_v,
             sem0, sem1, gsem):
  cid = lax.axis_index("c")
  sid = lax.axis_index("s")
  wid = sid * _NC + cid

  pltpu.sync_copy(ij_hbm, idx_v)
  idx = idx_v[...]
  ii = idx[0]
  jj = idx[1]
  base_i = pl.multiple_of((ii // _LANE) * _LANE, _LANE)
  base_j = pl.multiple_of((jj // _LANE) * _LANE, _LANE)
  pltpu.sync_copy(tt_hbm.at[:, pl.ds(base_i, _LANE)], tile_i)
  pltpu.sync_copy(tt_hbm.at[:, pl.ds(base_j, _LANE)], tile_j)
  col_i = jnp.full((_N_DIM,), ii % _LANE, jnp.int32)
  col_j = jnp.full((_N_DIM,), jj % _LANE, jnp.int32)
  dim_iota = lax.iota(jnp.int32, _N_DIM)
  tis = tuple(
      plsc.load_gather(tile_i, [jnp.full((_N_DIM,), d, jnp.int32), col_i])
      for d in range(_N_DIM))
  rows_v[0, :] = plsc.load_gather(tile_i, [dim_iota, col_i])
  rows_v[1, :] = plsc.load_gather(tile_j, [dim_iota, col_j])

  bufs = (buf0, buf1)
  sems = (sem0, sem1)

  def start(c):
    g = c * _NW + wid
    if c == _CPW - 1:
      g = jnp.where(wid < _FULL_W, g, 0)
    off = pl.multiple_of(g * _CHW, _LANE)
    return pltpu.async_copy(
        tt_hbm.at[:, pl.ds(off, _CHW)], bufs[c % 2], sems[c % 2])

  def chunk_sum(buf, accs):
    @pl.loop(0, _CHW // _N_DIM, init_carry=accs)
    def accs(l, carry):
      off = l * _N_DIM
      out = []
      for d in range(_N_DIM):
        x = buf[d, pl.ds(off, _N_DIM)]
        dd = x - tis[d]
        out.append(carry[d] + jnp.exp(-(dd * dd)))
      return tuple(out)
    return accs

  zeros = tuple(jnp.zeros((_N_DIM,), jnp.float32) for _ in range(_N_DIM))
  inflight = start(0)
  accs = zeros
  for c in range(_CPW - 1):
    inflight.wait()
    nxt = start(c + 1)
    accs = chunk_sum(bufs[c % 2], accs)
    inflight = nxt
  inflight.wait()
  extra = chunk_sum(bufs[(_CPW - 1) % 2], zeros)

  zero_v = jnp.zeros((_N_DIM,), jnp.float32)
  for d in range(_N_DIM):
    acc_v[d, :] = accs[d] + jnp.where(wid < _FULL_W, extra[d], zero_v)
  pltpu.sync_copy(acc_v, partials_hbm.at[wid])

  @pl.when(wid == 0)
  def _():
    pltpu.sync_copy(rows_v, rows_hbm)


@functools.partial(
    pl.kernel,
    out_type=(
        jax.ShapeDtypeStruct((_NW, _N_DIM, _N_DIM), jnp.float32),
        jax.ShapeDtypeStruct((2, _N_DIM), jnp.float32),
    ),
    mesh=plsc.VectorSubcoreMesh(core_axis_name="c", subcore_axis_name="s"),
    compiler_params=pltpu.CompilerParams(
        use_tc_tiling_on_sc=True, needs_layout_passes=False),
    scratch_types=(
        pltpu.VMEM((_N_DIM,), jnp.int32),
        pltpu.VMEM((_N_DIM, _LANE), jnp.float32),
        pltpu.VMEM((_N_DIM, _LANE), jnp.float32),
        pltpu.VMEM((2, _N_DIM), jnp.float32),
        pltpu.VMEM((_N_DIM, _CHW), jnp.float32),
        pltpu.VMEM((_N_DIM, _CHW), jnp.float32),
        pltpu.VMEM((_N_DIM, _N_DIM), jnp.float32),
        pltpu.SemaphoreType.DMA,
        pltpu.SemaphoreType.DMA,
        pltpu.SemaphoreType.DMA,
    ),
)
def _sc_reduce(tt_hbm, ij_hbm, partials_hbm, rows_hbm, *scratch):
  _sc_body(tt_hbm, ij_hbm, partials_hbm, rows_hbm, *scratch)


def _tc_epilogue(pij_ref, rows_ref, partials_ref, tail_ref, out_ref):
  ti = rows_ref[0:1, :]
  tj = rows_ref[1:2, :]
  q_sc = jnp.sum(partials_ref[...])
  dt = tail_ref[...] - ti
  q_tail = jnp.sum(jnp.exp(-(dt * dt)))
  q_sum = q_sc + q_tail - jnp.float32(_N_DIM)
  d = ti - tj
  p = pij_ref[...]
  t = p * (jnp.log(p) + d * d + jnp.log(q_sum))
  out_ref[...] = jnp.sum(t).reshape(1, 1)


def kernel(pij, i, j, table):
  ij = jnp.concatenate(
      [i.astype(jnp.int32), j.astype(jnp.int32),
       jnp.zeros((_N_DIM - 2,), jnp.int32)])
  tt = table.T
  partials, rows = _sc_reduce(tt, ij)
  tail = lax.slice(table, (_NT_FULL * _LANE, 0), (_N_POINTS, _N_DIM))
  out = pl.pallas_call(
      _tc_epilogue,
      out_shape=jax.ShapeDtypeStruct((1, 1), jnp.float32),
  )(pij.reshape(1, _N_DIM), rows, partials, tail)
  return out[0, 0]
